# Initial kernel scaffold; baseline (speedup 1.0000x reference)
import jax
import jax.numpy as jnp
from jax.experimental import pallas as pl


def kernel(batch_embedding, batch_target):
    raise NotImplementedError("write your pallas kernel here")


# speedup vs baseline: 3.9619x; 3.9619x over previous
import jax
import jax.numpy as jnp
from jax.experimental import pallas as pl
from jax.experimental.pallas import tpu as pltpu

N = 262144
D = 32
C = 32
BN = 8192
NBLK = N // BN
DELTA_VAR = 0.5
DELTA_D = 1.5
GAMMA = 0.001
_HI = jax.lax.Precision.HIGHEST


def _dotg(a, b):
    return jax.lax.dot_general(a, b, (((1,), (1,)), ((), ())), precision=_HI,
                               preferred_element_type=jnp.float32)


def _body(x_ref, trow_ref, out_ref, sums_ref, counts_ref, hinge_ref,
          means_ref):
    i = pl.program_id(0)

    @pl.when(i == 0)
    def _init():
        sums_ref[...] = jnp.zeros_like(sums_ref)
        counts_ref[...] = jnp.zeros_like(counts_ref)
        hinge_ref[...] = jnp.zeros_like(hinge_ref)

    x = x_ref[0]
    t_row = trow_ref[0]
    iota_c = jax.lax.broadcasted_iota(jnp.int32, (C, 1), 0)
    oh_row = (t_row == iota_c).astype(jnp.float32)

    @pl.when(i < NBLK)
    def _pass1():
        sums_ref[...] += jax.lax.dot(oh_row, x, precision=_HI,
                                     preferred_element_type=jnp.float32)
        counts_ref[...] += jnp.sum(oh_row, axis=1, keepdims=True)

    @pl.when(i == NBLK - 1)
    def _means():
        safe = jnp.maximum(counts_ref[...], 1.0)
        means_ref[...] = sums_ref[...] / safe

    @pl.when(i >= NBLK)
    def _pass2():
        m = means_ref[...]
        pt = _dotg(m, x)
        sel = jnp.sum(oh_row * pt, axis=0, keepdims=True)
        mnrm = jnp.sum(m * m, axis=1, keepdims=True)
        mnrm_sel = jnp.sum(oh_row * mnrm, axis=0, keepdims=True)
        ones_row = jnp.ones((1, D), jnp.float32)
        xnrm = _dotg(ones_row, x * x)
        d2 = xnrm - 2.0 * sel + mnrm_sel
        dist = jnp.sqrt(jnp.maximum(d2, 0.0) + 1e-12)
        h = jnp.maximum(dist - DELTA_VAR, 0.0)
        hinge_ref[...] += _dotg(oh_row, h * h)

    @pl.when(i == 2 * NBLK - 1)
    def _final():
        safe = jnp.maximum(counts_ref[...], 1.0)
        var_term = jnp.sum(hinge_ref[...] / safe) / C
        m = means_ref[...]
        gram = _dotg(m, m)
        ii = jax.lax.broadcasted_iota(jnp.int32, (C, C), 0)
        jj = jax.lax.broadcasted_iota(jnp.int32, (C, C), 1)
        eye = (ii == jj).astype(jnp.float32)
        diag_row = jnp.sum(gram * eye, axis=0, keepdims=True)
        diag_col = jnp.sum(gram * eye, axis=1, keepdims=True)
        pd2 = jnp.maximum(diag_col + diag_row - 2.0 * gram, 0.0)
        pd = jnp.sqrt(pd2 + 1e-12)
        dh = jnp.maximum(2.0 * DELTA_D - pd, 0.0)
        distance_term = jnp.sum(dh * dh * (1.0 - eye)) / (C * (C - 1))
        reg = jnp.sum(jnp.sqrt(diag_row + 1e-12)) / C
        total = var_term + distance_term + GAMMA * reg
        out_ref[...] = jnp.broadcast_to(total, (1, 1))


def kernel(batch_embedding, batch_target):
    t = batch_target.astype(jnp.int32)
    trow = t.reshape(2 * NBLK, 1, BN)
    res = pl.pallas_call(
        _body,
        grid=(2 * NBLK,),
        in_specs=[
            pl.BlockSpec((1, BN, D), lambda i: (0, i % NBLK, 0)),
            pl.BlockSpec((1, 1, BN), lambda i: (i % NBLK, 0, 0)),
        ],
        out_specs=pl.BlockSpec((1, 1), lambda i: (0, 0)),
        out_shape=jax.ShapeDtypeStruct((1, 1), jnp.float32),
        scratch_shapes=[
            pltpu.VMEM((C, D), jnp.float32),
            pltpu.VMEM((C, 1), jnp.float32),
            pltpu.VMEM((C, 1), jnp.float32),
            pltpu.VMEM((C, D), jnp.float32),
        ],
    )(batch_embedding, trow)
    return res[0, 0]


# speedup vs baseline: 6.7106x; 1.6938x over previous
import jax
import jax.numpy as jnp
from jax.experimental import pallas as pl
from jax.experimental.pallas import tpu as pltpu

N = 262144
D = 32
C = 32
BN = 8192
NBLK = N // BN
DELTA_VAR = 0.5
DELTA_D = 1.5
GAMMA = 0.001
_HI = jax.lax.Precision.HIGHEST


def _dotg(a, b, prec=None):
    return jax.lax.dot_general(a, b, (((1,), (1,)), ((), ())), precision=prec,
                               preferred_element_type=jnp.float32)


def _body(x_ref, trow_ref, out_ref, sums_ref, counts_ref, hinge_ref,
          means_ref):
    i = pl.program_id(0)

    @pl.when(i == 0)
    def _init():
        sums_ref[...] = jnp.zeros_like(sums_ref)
        counts_ref[...] = jnp.zeros_like(counts_ref)
        hinge_ref[...] = jnp.zeros_like(hinge_ref)

    x = x_ref[0]
    t_row = trow_ref[0]
    iota_c = jax.lax.broadcasted_iota(jnp.int32, (C, 1), 0)
    mask = t_row == iota_c
    oh16 = mask.astype(jnp.bfloat16)
    xh = x.astype(jnp.bfloat16)

    @pl.when(i < NBLK)
    def _pass1():
        xl = (x - xh.astype(jnp.float32)).astype(jnp.bfloat16)
        sums_ref[...] += (jax.lax.dot(oh16, xh,
                                      preferred_element_type=jnp.float32)
                          + jax.lax.dot(oh16, xl,
                                        preferred_element_type=jnp.float32))
        counts_ref[...] += jnp.sum(mask.astype(jnp.float32), axis=1,
                                   keepdims=True)

    @pl.when(i == NBLK - 1)
    def _means():
        safe = jnp.maximum(counts_ref[...], 1.0)
        means_ref[...] = sums_ref[...] / safe

    @pl.when(i >= NBLK)
    def _pass2():
        m = means_ref[...]
        m16 = m.astype(jnp.bfloat16)
        pt = _dotg(m16, xh)
        mnrm = jnp.sum(m * m, axis=1, keepdims=True)
        pt2 = pt - 0.5 * mnrm
        sel2 = jnp.sum(jnp.where(mask, pt2, 0.0), axis=0,
                       keepdims=True)
        x2h = (x * x).astype(jnp.bfloat16)
        ones_d = jnp.ones((1, D), jnp.bfloat16)
        xnrm = _dotg(ones_d, x2h)
        d2 = xnrm - 2.0 * sel2
        dist = jnp.sqrt(jnp.maximum(d2, 0.0) + 1e-12)
        h = jnp.maximum(dist - DELTA_VAR, 0.0)
        hinge_ref[...] += _dotg(mask.astype(jnp.float32), h * h)

    @pl.when(i == 2 * NBLK - 1)
    def _final():
        safe = jnp.maximum(counts_ref[...], 1.0)
        var_term = jnp.sum(hinge_ref[...] / safe) / C
        m = means_ref[...]
        gram = _dotg(m, m, _HI)
        ii = jax.lax.broadcasted_iota(jnp.int32, (C, C), 0)
        jj = jax.lax.broadcasted_iota(jnp.int32, (C, C), 1)
        eye = (ii == jj).astype(jnp.float32)
        diag_row = jnp.sum(gram * eye, axis=0, keepdims=True)
        diag_col = jnp.sum(gram * eye, axis=1, keepdims=True)
        pd2 = jnp.maximum(diag_col + diag_row - 2.0 * gram, 0.0)
        pd = jnp.sqrt(pd2 + 1e-12)
        dh = jnp.maximum(2.0 * DELTA_D - pd, 0.0)
        distance_term = jnp.sum(dh * dh * (1.0 - eye)) / (C * (C - 1))
        reg = jnp.sum(jnp.sqrt(diag_row + 1e-12)) / C
        total = var_term + distance_term + GAMMA * reg
        out_ref[...] = jnp.broadcast_to(total, (1, 1))


def kernel(batch_embedding, batch_target):
    t = batch_target.astype(jnp.int32)
    trow = t.reshape(2 * NBLK, 1, BN)
    res = pl.pallas_call(
        _body,
        grid=(2 * NBLK,),
        in_specs=[
            pl.BlockSpec((1, BN, D), lambda i: (0, i % NBLK, 0)),
            pl.BlockSpec((1, 1, BN), lambda i: (i % NBLK, 0, 0)),
        ],
        out_specs=pl.BlockSpec((1, 1), lambda i: (0, 0)),
        out_shape=jax.ShapeDtypeStruct((1, 1), jnp.float32),
        scratch_shapes=[
            pltpu.VMEM((C, D), jnp.float32),
            pltpu.VMEM((C, 1), jnp.float32),
            pltpu.VMEM((C, 1), jnp.float32),
            pltpu.VMEM((C, D), jnp.float32),
        ],
    )(batch_embedding, trow)
    return res[0, 0]


# speedup vs baseline: 7.2378x; 1.0786x over previous
import jax
import jax.numpy as jnp
from jax.experimental import pallas as pl
from jax.experimental.pallas import tpu as pltpu

N = 262144
D = 32
C = 32
BN = 16384
NBLK = N // BN
DELTA_VAR = 0.5
DELTA_D = 1.5
GAMMA = 0.001
_HI = jax.lax.Precision.HIGHEST


def _dotg(a, b, prec=None):
    return jax.lax.dot_general(a, b, (((1,), (1,)), ((), ())), precision=prec,
                               preferred_element_type=jnp.float32)


def _body(x_ref, trow_ref, out_ref, sums_ref, counts_ref, hinge_ref,
          means_ref):
    i = pl.program_id(0)

    @pl.when(i == 0)
    def _init():
        sums_ref[...] = jnp.zeros_like(sums_ref)
        counts_ref[...] = jnp.zeros_like(counts_ref)
        hinge_ref[...] = jnp.zeros_like(hinge_ref)

    x = x_ref[0]
    t_row = trow_ref[0]
    iota_c = jax.lax.broadcasted_iota(jnp.int32, (C, 1), 0)
    mask = t_row == iota_c
    oh16 = mask.astype(jnp.bfloat16)
    xh = x.astype(jnp.bfloat16)

    @pl.when(i < NBLK)
    def _pass1():
        sums_ref[...] += jax.lax.dot(oh16, xh,
                                     preferred_element_type=jnp.float32)
        ones_row = (jax.lax.broadcasted_iota(jnp.int32, (8, BN), 0)
                    >= 0).astype(jnp.bfloat16)
        counts_ref[...] += _dotg(oh16, ones_row)

    @pl.when(i == NBLK - 1)
    def _means():
        safe = jnp.maximum(counts_ref[:, 0:1], 1.0)
        means_ref[...] = sums_ref[...] / safe

    @pl.when(i >= NBLK)
    def _pass2():
        m = means_ref[...]
        m16 = m.astype(jnp.bfloat16)
        pt = _dotg(m16, xh)
        mnrm = jnp.sum(m * m, axis=1, keepdims=True)
        pt2 = pt - 0.5 * mnrm
        sel2 = jnp.sum(jnp.where(mask, pt2, 0.0), axis=0,
                       keepdims=True)
        x2h = (x * x).astype(jnp.bfloat16)
        ones_d = jnp.ones((1, D), jnp.bfloat16)
        xnrm = _dotg(ones_d, x2h)
        d2 = xnrm - 2.0 * sel2
        dist = jnp.sqrt(jnp.maximum(d2, 0.0) + 1e-12)
        h = jnp.maximum(dist - DELTA_VAR, 0.0)
        hh8 = jnp.broadcast_to(h * h, (8, BN)).astype(jnp.bfloat16)
        hinge_ref[...] += _dotg(oh16, hh8)

    @pl.when(i == 2 * NBLK - 1)
    def _final():
        safe = jnp.maximum(counts_ref[:, 0:1], 1.0)
        var_term = jnp.sum(hinge_ref[:, 0:1] / safe) / C
        m = means_ref[...]
        gram = _dotg(m, m, _HI)
        ii = jax.lax.broadcasted_iota(jnp.int32, (C, C), 0)
        jj = jax.lax.broadcasted_iota(jnp.int32, (C, C), 1)
        eye = (ii == jj).astype(jnp.float32)
        diag_row = jnp.sum(gram * eye, axis=0, keepdims=True)
        diag_col = jnp.sum(gram * eye, axis=1, keepdims=True)
        pd2 = jnp.maximum(diag_col + diag_row - 2.0 * gram, 0.0)
        pd = jnp.sqrt(pd2 + 1e-12)
        dh = jnp.maximum(2.0 * DELTA_D - pd, 0.0)
        distance_term = jnp.sum(dh * dh * (1.0 - eye)) / (C * (C - 1))
        reg = jnp.sum(jnp.sqrt(diag_row + 1e-12)) / C
        total = var_term + distance_term + GAMMA * reg
        out_ref[...] = jnp.broadcast_to(total, (1, 1))


def kernel(batch_embedding, batch_target):
    t = batch_target.astype(jnp.int32)
    trow = t.reshape(2 * NBLK, 1, BN)
    res = pl.pallas_call(
        _body,
        grid=(2 * NBLK,),
        in_specs=[
            pl.BlockSpec((1, BN, D), lambda i: (0, i % NBLK, 0)),
            pl.BlockSpec((1, 1, BN), lambda i: (i % NBLK, 0, 0)),
        ],
        out_specs=pl.BlockSpec((1, 1), lambda i: (0, 0)),
        out_shape=jax.ShapeDtypeStruct((1, 1), jnp.float32),
        scratch_shapes=[
            pltpu.VMEM((C, D), jnp.float32),
            pltpu.VMEM((C, 8), jnp.float32),
            pltpu.VMEM((C, 8), jnp.float32),
            pltpu.VMEM((C, D), jnp.float32),
        ],
    )(batch_embedding, trow)
    return res[0, 0]


# speedup vs baseline: 13.6916x; 1.8917x over previous
import jax
import jax.numpy as jnp
from jax.experimental import pallas as pl
from jax.experimental.pallas import tpu as pltpu

N = 262144
D = 32
C = 32
BN = 16384
NBLK = N // BN
DELTA_VAR = 0.5
DELTA_D = 1.5
GAMMA = 0.001
_HI = jax.lax.Precision.HIGHEST


def _dotg(a, b, prec=None):
    return jax.lax.dot_general(a, b, (((1,), (1,)), ((), ())), precision=prec,
                               preferred_element_type=jnp.float32)


def _body(x_ref, trow_ref, out_ref, sums_ref, counts_ref, hinge_ref,
          means_ref):
    i = pl.program_id(0)

    @pl.when(i == 0)
    def _init():
        sums_ref[...] = jnp.zeros_like(sums_ref)
        counts_ref[...] = jnp.zeros_like(counts_ref)
        hinge_ref[...] = jnp.zeros_like(hinge_ref)

    xh = x_ref[...]
    t_row = trow_ref[0]
    iota_c = jax.lax.broadcasted_iota(jnp.int32, (C, 1), 0)
    mask = t_row == iota_c
    oh16 = mask.astype(jnp.bfloat16)

    @pl.when(i < NBLK)
    def _pass1():
        sums_ref[...] += jax.lax.dot(oh16, xh,
                                     preferred_element_type=jnp.float32)
        ones_row = (jax.lax.broadcasted_iota(jnp.int32, (8, BN), 0)
                    >= 0).astype(jnp.bfloat16)
        counts_ref[...] += _dotg(oh16, ones_row)

    @pl.when(i == NBLK - 1)
    def _means():
        safe = jnp.maximum(counts_ref[:, 0:1], 1.0)
        means_ref[...] = sums_ref[...] / safe

    @pl.when(i >= NBLK)
    def _pass2():
        m = means_ref[...]
        m16 = m.astype(jnp.bfloat16)
        pt = _dotg(m16, xh)
        mnrm = jnp.sum(m * m, axis=1, keepdims=True)
        pt2 = pt - 0.5 * mnrm
        sel2 = jnp.sum(jnp.where(mask, pt2, 0.0), axis=0,
                       keepdims=True)
        x2h = xh * xh
        ones_d = jnp.ones((1, D), jnp.bfloat16)
        xnrm = _dotg(ones_d, x2h)
        d2 = xnrm - 2.0 * sel2
        dist = jnp.sqrt(jnp.maximum(d2, 0.0) + 1e-12)
        h = jnp.maximum(dist - DELTA_VAR, 0.0)
        hh8 = jnp.broadcast_to(h * h, (8, BN)).astype(jnp.bfloat16)
        hinge_ref[...] += _dotg(oh16, hh8)

    @pl.when(i == 2 * NBLK - 1)
    def _final():
        safe = jnp.maximum(counts_ref[:, 0:1], 1.0)
        var_term = jnp.sum(hinge_ref[:, 0:1] / safe) / C
        m = means_ref[...]
        gram = _dotg(m, m, _HI)
        ii = jax.lax.broadcasted_iota(jnp.int32, (C, C), 0)
        jj = jax.lax.broadcasted_iota(jnp.int32, (C, C), 1)
        eye = (ii == jj).astype(jnp.float32)
        diag_row = jnp.sum(gram * eye, axis=0, keepdims=True)
        diag_col = jnp.sum(gram * eye, axis=1, keepdims=True)
        pd2 = jnp.maximum(diag_col + diag_row - 2.0 * gram, 0.0)
        pd = jnp.sqrt(pd2 + 1e-12)
        dh = jnp.maximum(2.0 * DELTA_D - pd, 0.0)
        distance_term = jnp.sum(dh * dh * (1.0 - eye)) / (C * (C - 1))
        reg = jnp.sum(jnp.sqrt(diag_row + 1e-12)) / C
        total = var_term + distance_term + GAMMA * reg
        out_ref[...] = jnp.broadcast_to(total, (1, 1))


def kernel(batch_embedding, batch_target):
    xbf = batch_embedding[0].astype(jnp.bfloat16)
    t = batch_target.astype(jnp.int32)
    trow = t.reshape(2 * NBLK, 1, BN)
    res = pl.pallas_call(
        _body,
        grid=(2 * NBLK,),
        in_specs=[
            pl.BlockSpec((BN, D), lambda i: (i % NBLK, 0)),
            pl.BlockSpec((1, 1, BN), lambda i: (i % NBLK, 0, 0)),
        ],
        out_specs=pl.BlockSpec((1, 1), lambda i: (0, 0)),
        out_shape=jax.ShapeDtypeStruct((1, 1), jnp.float32),
        scratch_shapes=[
            pltpu.VMEM((C, D), jnp.float32),
            pltpu.VMEM((C, 8), jnp.float32),
            pltpu.VMEM((C, 8), jnp.float32),
            pltpu.VMEM((C, D), jnp.float32),
        ],
    )(xbf, trow)
    return res[0, 0]


# speedup vs baseline: 30.9654x; 2.2616x over previous
import jax
import jax.numpy as jnp
from jax.experimental import pallas as pl
from jax.experimental.pallas import tpu as pltpu

N = 262144
D = 32
C = 32
BN = 16384
NBLK = N // BN
DELTA_VAR = 0.5
DELTA_D = 1.5
GAMMA = 0.001
_HI = jax.lax.Precision.HIGHEST


def _dotg(a, b, prec=None):
    return jax.lax.dot_general(a, b, (((1,), (1,)), ((), ())), precision=prec,
                               preferred_element_type=jnp.float32)


def _ones_bf16(shape):
    return (jax.lax.broadcasted_iota(jnp.int32, shape, 0)
            >= 0).astype(jnp.bfloat16)


def _body(x_ref, trow_ref, out_ref, sums_ref, counts_ref, hinge_ref,
          means_ref, means16_ref, xc_ref):
    i = pl.program_id(0)
    jb = jax.lax.rem(i, NBLK)

    @pl.when(i == 0)
    def _init():
        sums_ref[...] = jnp.zeros_like(sums_ref)
        counts_ref[...] = jnp.zeros_like(counts_ref)
        hinge_ref[...] = jnp.zeros_like(hinge_ref)

    t_row = trow_ref[0]
    iota_c = jax.lax.broadcasted_iota(jnp.int32, (C, 1), 0)
    mask = t_row == iota_c
    oh16 = mask.astype(jnp.bfloat16)

    @pl.when(i < NBLK)
    def _pass1():
        xt = x_ref[...]
        xc_ref[jb] = xt
        sums_ref[...] += _dotg(oh16, xt)
        counts_ref[...] += _dotg(oh16, _ones_bf16((8, BN)))

    @pl.when(i == NBLK - 1)
    def _means():
        safe = jnp.maximum(counts_ref[:, 0:1], 1.0)
        means = sums_ref[...] / safe
        means_ref[...] = means
        means16_ref[...] = means.astype(jnp.bfloat16)

    @pl.when(i >= NBLK)
    def _pass2():
        xt = xc_ref[jb]
        m = means_ref[...]
        pt = jax.lax.dot(means16_ref[...], xt,
                         preferred_element_type=jnp.float32)
        mnrm = jnp.sum(m * m, axis=1, keepdims=True)
        pt2 = pt - 0.5 * mnrm
        sel2 = jnp.sum(jnp.where(mask, pt2, 0.0), axis=0,
                       keepdims=True)
        xnrm8 = jax.lax.dot(_ones_bf16((8, D)), xt * xt,
                            preferred_element_type=jnp.float32)
        d2 = xnrm8[0:1] - 2.0 * sel2
        dist = jnp.sqrt(jnp.maximum(d2, 0.0) + 1e-12)
        h = jnp.maximum(dist - DELTA_VAR, 0.0)
        hh8 = jnp.broadcast_to(h * h, (8, BN)).astype(jnp.bfloat16)
        hinge_ref[...] += _dotg(oh16, hh8)

    @pl.when(i == 2 * NBLK - 1)
    def _final():
        safe = jnp.maximum(counts_ref[:, 0:1], 1.0)
        var_term = jnp.sum(hinge_ref[:, 0:1] / safe) / C
        m = means_ref[...]
        gram = _dotg(m, m, _HI)
        ii = jax.lax.broadcasted_iota(jnp.int32, (C, C), 0)
        jj = jax.lax.broadcasted_iota(jnp.int32, (C, C), 1)
        eye = (ii == jj).astype(jnp.float32)
        diag_row = jnp.sum(gram * eye, axis=0, keepdims=True)
        diag_col = jnp.sum(gram * eye, axis=1, keepdims=True)
        pd2 = jnp.maximum(diag_col + diag_row - 2.0 * gram, 0.0)
        pd = jnp.sqrt(pd2 + 1e-12)
        dh = jnp.maximum(2.0 * DELTA_D - pd, 0.0)
        distance_term = jnp.sum(dh * dh * (1.0 - eye)) / (C * (C - 1))
        reg = jnp.sum(jnp.sqrt(diag_row + 1e-12)) / C
        total = var_term + distance_term + GAMMA * reg
        out_ref[...] = jnp.broadcast_to(total, (1, 1))


def kernel(batch_embedding, batch_target):
    xt = batch_embedding[0].astype(jnp.bfloat16).T
    t = batch_target.astype(jnp.int32)
    trow = t.reshape(2 * NBLK, 1, BN)
    res = pl.pallas_call(
        _body,
        grid=(2 * NBLK,),
        in_specs=[
            pl.BlockSpec((D, BN), lambda i: (0, jnp.where(i < NBLK, i, 0))),
            pl.BlockSpec((1, 1, BN), lambda i: (i % NBLK, 0, 0)),
        ],
        out_specs=pl.BlockSpec((1, 1), lambda i: (0, 0)),
        out_shape=jax.ShapeDtypeStruct((1, 1), jnp.float32),
        scratch_shapes=[
            pltpu.VMEM((C, D), jnp.float32),
            pltpu.VMEM((C, 8), jnp.float32),
            pltpu.VMEM((C, 8), jnp.float32),
            pltpu.VMEM((C, D), jnp.float32),
            pltpu.VMEM((C, D), jnp.bfloat16),
            pltpu.VMEM((NBLK, D, BN), jnp.bfloat16),
        ],
    )(xt, trow)
    return res[0, 0]


# speedup vs baseline: 33.9738x; 1.0972x over previous
import jax
import jax.numpy as jnp
from jax.experimental import pallas as pl
from jax.experimental.pallas import tpu as pltpu

N = 262144
D = 32
C = 32
BN = 32768
NBLK = N // BN
DELTA_VAR = 0.5
DELTA_D = 1.5
GAMMA = 0.001
_HI = jax.lax.Precision.HIGHEST


def _dotg(a, b, prec=None):
    return jax.lax.dot_general(a, b, (((1,), (1,)), ((), ())), precision=prec,
                               preferred_element_type=jnp.float32)


def _ones_bf16(shape):
    return (jax.lax.broadcasted_iota(jnp.int32, shape, 0)
            >= 0).astype(jnp.bfloat16)


def _body(x_ref, trow_ref, out_ref, sums_ref, counts_ref, hinge_ref,
          means_ref, means16_ref, xc_ref):
    i = pl.program_id(0)
    jb = jax.lax.rem(i, NBLK)

    @pl.when(i == 0)
    def _init():
        sums_ref[...] = jnp.zeros_like(sums_ref)
        counts_ref[...] = jnp.zeros_like(counts_ref)
        hinge_ref[...] = jnp.zeros_like(hinge_ref)

    t_row = trow_ref[0]
    iota_c = jax.lax.broadcasted_iota(jnp.int32, (C, 1), 0)
    mask = t_row == iota_c
    oh16 = mask.astype(jnp.bfloat16)

    @pl.when(i < NBLK)
    def _pass1():
        xt = x_ref[...]
        xc_ref[jb] = xt
        sums_ref[...] += _dotg(oh16, xt)
        counts_ref[...] += _dotg(oh16, _ones_bf16((8, BN)))

    @pl.when(i == NBLK - 1)
    def _means():
        safe = jnp.maximum(counts_ref[:, 0:1], 1.0)
        means = sums_ref[...] / safe
        means_ref[...] = means
        means16_ref[...] = means.astype(jnp.bfloat16)

    @pl.when(i >= NBLK)
    def _pass2():
        xt = xc_ref[jb]
        m = means_ref[...]
        pt = jax.lax.dot(means16_ref[...], xt,
                         preferred_element_type=jnp.float32)
        mnrm = jnp.sum(m * m, axis=1, keepdims=True)
        pt2 = pt - 0.5 * mnrm
        sel2 = jnp.sum(jnp.where(mask, pt2, 0.0), axis=0,
                       keepdims=True)
        xnrm8 = jax.lax.dot(_ones_bf16((8, D)), xt * xt,
                            preferred_element_type=jnp.float32)
        d2 = xnrm8[0:1] - 2.0 * sel2
        dist = jnp.sqrt(jnp.maximum(d2, 0.0) + 1e-12)
        h = jnp.maximum(dist - DELTA_VAR, 0.0)
        hh8 = jnp.broadcast_to(h * h, (8, BN)).astype(jnp.bfloat16)
        hinge_ref[...] += _dotg(oh16, hh8)

    @pl.when(i == 2 * NBLK - 1)
    def _final():
        safe = jnp.maximum(counts_ref[:, 0:1], 1.0)
        var_term = jnp.sum(hinge_ref[:, 0:1] / safe) / C
        m = means_ref[...]
        gram = _dotg(m, m, _HI)
        ii = jax.lax.broadcasted_iota(jnp.int32, (C, C), 0)
        jj = jax.lax.broadcasted_iota(jnp.int32, (C, C), 1)
        eye = (ii == jj).astype(jnp.float32)
        diag_row = jnp.sum(gram * eye, axis=0, keepdims=True)
        diag_col = jnp.sum(gram * eye, axis=1, keepdims=True)
        pd2 = jnp.maximum(diag_col + diag_row - 2.0 * gram, 0.0)
        pd = jnp.sqrt(pd2 + 1e-12)
        dh = jnp.maximum(2.0 * DELTA_D - pd, 0.0)
        distance_term = jnp.sum(dh * dh * (1.0 - eye)) / (C * (C - 1))
        reg = jnp.sum(jnp.sqrt(diag_row + 1e-12)) / C
        total = var_term + distance_term + GAMMA * reg
        out_ref[...] = jnp.broadcast_to(total, (1, 1))


def kernel(batch_embedding, batch_target):
    xt = batch_embedding[0].astype(jnp.bfloat16).T
    t = batch_target.astype(jnp.int32)
    trow = t.reshape(2 * NBLK, 1, BN)
    res = pl.pallas_call(
        _body,
        grid=(2 * NBLK,),
        in_specs=[
            pl.BlockSpec((D, BN), lambda i: (0, jnp.where(i < NBLK, i, 0))),
            pl.BlockSpec((1, 1, BN), lambda i: (i % NBLK, 0, 0)),
        ],
        out_specs=pl.BlockSpec((1, 1), lambda i: (0, 0)),
        out_shape=jax.ShapeDtypeStruct((1, 1), jnp.float32),
        scratch_shapes=[
            pltpu.VMEM((C, D), jnp.float32),
            pltpu.VMEM((C, 8), jnp.float32),
            pltpu.VMEM((C, 8), jnp.float32),
            pltpu.VMEM((C, D), jnp.float32),
            pltpu.VMEM((C, D), jnp.bfloat16),
            pltpu.VMEM((NBLK, D, BN), jnp.bfloat16),
        ],
    )(xt, trow)
    return res[0, 0]
